# baseline (speedup 1.0000x reference)
import functools

import jax
import jax.numpy as jnp
from jax import lax
from jax.experimental import pallas as pl
from jax.experimental.pallas import tpu as pltpu
from jax.experimental.pallas import tpu_sc as plsc

_NUM_ROWS = 100000
_DIM = 64
_BATCH = 16384

_info = plsc.get_sparse_core_info()
_NC, _NS = _info.num_cores, _info.num_subcores
_NW = _NC * _NS
_B_PER_W = _BATCH // _NW
_CHUNK = 128
_NCHUNK = _B_PER_W // _CHUNK

_mesh = plsc.VectorSubcoreMesh(core_axis_name="c", subcore_axis_name="s")


@functools.partial(
    pl.kernel,
    mesh=_mesh,
    out_type=jax.ShapeDtypeStruct((_BATCH, _DIM), jnp.float32),
    scratch_types=[
        pltpu.VMEM((_NCHUNK, _CHUNK), jnp.int32),
        pltpu.VMEM((_NCHUNK, _CHUNK, _DIM), jnp.float32),
        pltpu.SemaphoreType.DMA,
        pltpu.SemaphoreType.DMA,
    ],
    compiler_params=pltpu.CompilerParams(use_tc_tiling_on_sc=False),
)
def _gather_kernel(table_hbm, idx_hbm, out_hbm, idx_v, rows_v, gsem, ssem):
    wid = lax.axis_index("s") * _NC + lax.axis_index("c")
    base = wid * _B_PER_W
    for j in range(_NCHUNK):
        pltpu.sync_copy(idx_hbm.at[pl.ds(base + j * _CHUNK, _CHUNK)], idx_v.at[j])
    gathers = [
        pltpu.async_copy(table_hbm.at[idx_v.at[j]], rows_v.at[j], gsem)
        for j in range(_NCHUNK)
    ]
    stores = []
    for j in range(_NCHUNK):
        gathers[j].wait()
        stores.append(
            pltpu.async_copy(
                rows_v.at[j], out_hbm.at[pl.ds(base + j * _CHUNK, _CHUNK)], ssem
            )
        )
    for s in stores:
        s.wait()


def kernel(style_id, embeddings):
    idx = style_id.astype(jnp.int32)
    return _gather_kernel(embeddings, idx)


# speedup vs baseline: 1.4827x; 1.4827x over previous
import functools

import jax
import jax.numpy as jnp
from jax import lax
from jax.experimental import pallas as pl
from jax.experimental.pallas import tpu as pltpu
from jax.experimental.pallas import tpu_sc as plsc

_NUM_ROWS = 100000
_DIM = 64
_BATCH = 16384

_info = plsc.get_sparse_core_info()
_NC, _NS = _info.num_cores, _info.num_subcores
_NW = _NC * _NS
_B_PER_W = _BATCH // _NW

_mesh = plsc.VectorSubcoreMesh(core_axis_name="c", subcore_axis_name="s")


@functools.partial(
    pl.kernel,
    mesh=_mesh,
    out_type=jax.ShapeDtypeStruct((_BATCH, _DIM), jnp.float32),
    scratch_types=[
        pltpu.VMEM((_B_PER_W,), jnp.int32),
        pltpu.VMEM((_B_PER_W, _DIM), jnp.float32),
        pltpu.SemaphoreType.DMA,
    ],
)
def _gather_kernel(table_hbm, idx_hbm, out_hbm, idx_v, rows_v, sem):
    wid = lax.axis_index("s") * _NC + lax.axis_index("c")
    base = wid * _B_PER_W
    pltpu.sync_copy(idx_hbm.at[pl.ds(base, _B_PER_W)], idx_v)

    def issue(g, carry):
        lanes = idx_v[pl.ds(g * 16, 16)]
        for k in range(16):
            row = lanes[k]
            pltpu.async_copy(
                table_hbm.at[pl.ds(row, 1)],
                rows_v.at[pl.ds(g * 16 + k, 1)],
                sem,
            )
        return carry

    lax.fori_loop(0, _B_PER_W // 16, issue, 0)

    def drain(r, carry):
        pltpu.make_async_copy(
            table_hbm.at[pl.ds(0, 1)], rows_v.at[pl.ds(0, 1)], sem
        ).wait()
        return carry

    lax.fori_loop(0, _B_PER_W, drain, 0)
    pltpu.sync_copy(rows_v, out_hbm.at[pl.ds(base, _B_PER_W)])


def kernel(style_id, embeddings):
    idx = style_id.astype(jnp.int32)
    return _gather_kernel(embeddings, idx)


# speedup vs baseline: 1.5220x; 1.0265x over previous
import functools

import jax
import jax.numpy as jnp
from jax import lax
from jax.experimental import pallas as pl
from jax.experimental.pallas import tpu as pltpu
from jax.experimental.pallas import tpu_sc as plsc

_NUM_ROWS = 100000
_ROWS_PAD = 100096
_DIM = 64
_BATCH = 16384

_info = plsc.get_sparse_core_info()
_NC, _NS = _info.num_cores, _info.num_subcores
_NW = _NC * _NS
_D_PER_W = _DIM // _NW
_CHUNK = 8192
_NCHUNK = _BATCH // _CHUNK

_mesh = plsc.VectorSubcoreMesh(core_axis_name="c", subcore_axis_name="s")


@functools.partial(
    pl.kernel,
    mesh=_mesh,
    out_type=jax.ShapeDtypeStruct((_DIM, _BATCH), jnp.float32),
    scratch_types=[
        pltpu.VMEM((_ROWS_PAD,), jnp.float32),
        pltpu.VMEM((_CHUNK,), jnp.int32),
        pltpu.VMEM((_CHUNK,), jnp.float32),
        pltpu.SemaphoreType.DMA,
    ],
    compiler_params=pltpu.CompilerParams(
        needs_layout_passes=False, disable_bounds_checks=True
    ),
)
def _gather_kernel(table_hbm, idx_hbm, out_hbm, seg_v, idx_v, val_v, sem):
    wid = lax.axis_index("s") * _NC + lax.axis_index("c")

    for t in range(_D_PER_W):
        d = wid * _D_PER_W + t
        pltpu.sync_copy(table_hbm.at[d], seg_v)
        for c in range(_NCHUNK):
            pltpu.sync_copy(idx_hbm.at[pl.ds(c * _CHUNK, _CHUNK)], idx_v)

            def body(i, carry):
                lanes = idx_v[pl.ds(i * 16, 16)]
                vals = plsc.load_gather(seg_v, [lanes])
                val_v[pl.ds(i * 16, 16)] = vals
                return carry

            lax.fori_loop(0, _CHUNK // 16, body, 0)
            pltpu.sync_copy(val_v, out_hbm.at[d, pl.ds(c * _CHUNK, _CHUNK)])


def kernel(style_id, embeddings):
    idx = style_id.astype(jnp.int32)
    table_t = jnp.pad(embeddings.T, ((0, 0), (0, _ROWS_PAD - _NUM_ROWS)))
    out_t = _gather_kernel(table_t, idx)
    return out_t.T


# speedup vs baseline: 1.7164x; 1.1277x over previous
import functools

import jax
import jax.numpy as jnp
from jax import lax
from jax.experimental import pallas as pl
from jax.experimental.pallas import tpu as pltpu
from jax.experimental.pallas import tpu_sc as plsc

_NUM_ROWS = 100000
_ROWS_PAD = 100096
_DIM = 64
_BATCH = 16384

_info = plsc.get_sparse_core_info()
_NC, _NS = _info.num_cores, _info.num_subcores
_NW = _NC * _NS
_D_PER_W = _DIM // _NW
_CHUNK = 8192
_NCHUNK = _BATCH // _CHUNK

_mesh = plsc.VectorSubcoreMesh(core_axis_name="c", subcore_axis_name="s")


@functools.partial(
    pl.kernel,
    mesh=_mesh,
    out_type=jax.ShapeDtypeStruct((_DIM, _BATCH), jnp.float32),
    scratch_types=[
        pltpu.VMEM((_ROWS_PAD,), jnp.float32),
        pltpu.VMEM((_BATCH,), jnp.int32),
        pltpu.VMEM((_CHUNK,), jnp.float32),
        pltpu.SemaphoreType.DMA,
    ],
    compiler_params=pltpu.CompilerParams(
        needs_layout_passes=False, disable_bounds_checks=True
    ),
)
def _gather_kernel(table_hbm, idx_hbm, out_hbm, seg_v, idx_v, val_v, sem):
    wid = lax.axis_index("s") * _NC + lax.axis_index("c")
    pltpu.sync_copy(idx_hbm, idx_v)

    _UNROLL = 8

    for t in range(_D_PER_W):
        d = wid * _D_PER_W + t
        pltpu.sync_copy(table_hbm.at[d], seg_v)
        for c in range(_NCHUNK):

            def body(i, carry):
                base = i * (16 * _UNROLL)
                for u in range(_UNROLL):
                    lanes = idx_v[pl.ds(c * _CHUNK + base + u * 16, 16)]
                    vals = plsc.load_gather(seg_v, [lanes])
                    val_v[pl.ds(base + u * 16, 16)] = vals
                return carry

            lax.fori_loop(0, _CHUNK // (16 * _UNROLL), body, 0)
            pltpu.sync_copy(val_v, out_hbm.at[d, pl.ds(c * _CHUNK, _CHUNK)])


def kernel(style_id, embeddings):
    idx = style_id.astype(jnp.int32)
    table_t = jnp.pad(embeddings.T, ((0, 0), (0, _ROWS_PAD - _NUM_ROWS)))
    out_t = _gather_kernel(table_t, idx)
    return out_t.T
